# Initial kernel scaffold; baseline (speedup 1.0000x reference)
import jax
import jax.numpy as jnp
from jax.experimental import pallas as pl


def kernel(x, w_gate, w_noise, expert_w, expert_b):
    raise NotImplementedError("write your pallas kernel here")


# speedup vs baseline: 3.5290x; 3.5290x over previous
import functools

import jax
import jax.numpy as jnp
from jax import lax
from jax.experimental import pallas as pl
from jax.experimental.pallas import tpu as pltpu
from jax.experimental.pallas import tpu_sc as plsc

NE = 16
DI = 1024
DH = 4096
N = 8192

TB = 512
BM = 256
P = N * 2 + NE * BM
NBLK = P // BM
NW = 32
TPW = N // NW
DCH = 64
CCH = 8


def _routing_body(x_ref, wg_ref, e0_ref, e1_ref, r0_ref, r1_ref,
                  g0_ref, g1_ref, cnt_ref, carry):
    @pl.when(pl.program_id(0) == 0)
    def _():
        carry[...] = jnp.zeros_like(carry)

    logits = jnp.dot(x_ref[...], wg_ref[...],
                     preferred_element_type=jnp.float32)
    col = lax.broadcasted_iota(jnp.int32, (TB, NE), 1)
    m0 = jnp.max(logits, axis=1, keepdims=True)
    i0 = jnp.min(jnp.where(logits == m0, col, NE), axis=1, keepdims=True)
    masked = jnp.where(col == i0, -jnp.inf, logits)
    m1 = jnp.max(masked, axis=1, keepdims=True)
    i1 = jnp.min(jnp.where(masked == m1, col, NE), axis=1, keepdims=True)
    e1x = jnp.exp(m1 - m0)
    den = 1.0 + e1x
    g0 = 1.0 / den
    g1 = e1x / den

    oh0 = (col == i0).astype(jnp.float32)
    oh1 = (col == i1).astype(jnp.float32)
    oh = oh0 + oh1
    r_io = lax.broadcasted_iota(jnp.int32, (TB, TB), 0)
    c_io = lax.broadcasted_iota(jnp.int32, (TB, TB), 1)
    tri = (c_io < r_io).astype(jnp.float32)
    csum = jnp.dot(tri, oh, preferred_element_type=jnp.float32)
    pre = csum + carry[...]
    r0 = jnp.sum(oh0 * pre, axis=1, keepdims=True)
    r1 = jnp.sum(oh1 * pre, axis=1, keepdims=True)
    new_carry = carry[...] + jnp.sum(oh, axis=0, keepdims=True)
    carry[...] = new_carry

    e0_ref[...] = i0
    e1_ref[...] = i1
    r0_ref[...] = r0.astype(jnp.int32)
    r1_ref[...] = r1.astype(jnp.int32)
    g0_ref[...] = g0
    g1_ref[...] = g1
    cnt_ref[...] = new_carry.astype(jnp.int32)


def _routing(x, w_gate, interpret=False):
    nblk = N // TB
    out_shapes = (
        jax.ShapeDtypeStruct((N, 1), jnp.int32),
        jax.ShapeDtypeStruct((N, 1), jnp.int32),
        jax.ShapeDtypeStruct((N, 1), jnp.int32),
        jax.ShapeDtypeStruct((N, 1), jnp.int32),
        jax.ShapeDtypeStruct((N, 1), jnp.float32),
        jax.ShapeDtypeStruct((N, 1), jnp.float32),
        jax.ShapeDtypeStruct((1, NE), jnp.int32),
    )
    tok_spec = pl.BlockSpec((TB, 1), lambda i: (i, 0))
    return pl.pallas_call(
        _routing_body,
        grid=(nblk,),
        in_specs=[
            pl.BlockSpec((TB, DI), lambda i: (i, 0)),
            pl.BlockSpec((DI, NE), lambda i: (0, 0)),
        ],
        out_specs=(tok_spec, tok_spec, tok_spec, tok_spec, tok_spec, tok_spec,
                   pl.BlockSpec((1, NE), lambda i: (0, 0))),
        out_shape=out_shapes,
        scratch_shapes=[pltpu.VMEM((1, NE), jnp.float32)],
        interpret=interpret,
    )(x, w_gate)


def _posmap_body(e0_ref, e1_ref, r0_ref, r1_ref, g0_ref, g1_ref, poff_ref,
                 p0_ref, p1_ref, g0r_ref, g1r_ref):
    col = lax.broadcasted_iota(jnp.int32, (TB, NE), 1)
    poff_f = poff_ref[...].astype(jnp.float32)
    oh0 = (col == e0_ref[...]).astype(jnp.float32)
    oh1 = (col == e1_ref[...]).astype(jnp.float32)
    p0 = r0_ref[...].astype(jnp.float32) + jnp.sum(oh0 * poff_f, axis=1,
                                                   keepdims=True)
    p1 = r1_ref[...].astype(jnp.float32) + jnp.sum(oh1 * poff_f, axis=1,
                                                   keepdims=True)
    p0_ref[...] = p0.astype(jnp.int32)
    p1_ref[...] = p1.astype(jnp.int32)
    g0r_ref[...] = jnp.broadcast_to(g0_ref[...], (TB, 128))
    g1r_ref[...] = jnp.broadcast_to(g1_ref[...], (TB, 128))


def _posmap(e0, e1, r0, r1, g0, g1, poff, interpret=False):
    tok_spec = pl.BlockSpec((TB, 1), lambda i: (i, 0))
    rep_spec = pl.BlockSpec((TB, 128), lambda i: (i, 0))
    return pl.pallas_call(
        _posmap_body,
        grid=(N // TB,),
        in_specs=[tok_spec, tok_spec, tok_spec, tok_spec, tok_spec, tok_spec,
                  pl.BlockSpec((1, NE), lambda i: (0, 0))],
        out_specs=(tok_spec, tok_spec, rep_spec, rep_spec),
        out_shape=(jax.ShapeDtypeStruct((N, 1), jnp.int32),
                   jax.ShapeDtypeStruct((N, 1), jnp.int32),
                   jax.ShapeDtypeStruct((N, 128), jnp.float32),
                   jax.ShapeDtypeStruct((N, 128), jnp.float32)),
        interpret=interpret,
    )(e0, e1, r0, r1, g0, g1, poff)


def _dispatch_body(x_hbm, p0_hbm, p1_hbm, g0_hbm, g1_hbm,
                   xs_hbm, gp_hbm,
                   pos0_v, pos1_v, rows_v, ga_v, gb_v, sem):
    w = lax.axis_index("s") * 2 + lax.axis_index("c")
    base = w * TPW
    nch = TPW // DCH
    for c in range(nch):
        cb = base + c * DCH
        pltpu.sync_copy(p0_hbm.at[pl.ds(cb, DCH)], pos0_v.at[c])
        pltpu.sync_copy(p1_hbm.at[pl.ds(cb, DCH)], pos1_v.at[c])
    for c in range(nch):
        cb = base + c * DCH
        pltpu.sync_copy(x_hbm.at[pl.ds(cb, DCH)], rows_v)
        pltpu.sync_copy(g0_hbm.at[pl.ds(cb, DCH), :], ga_v)
        pltpu.sync_copy(g1_hbm.at[pl.ds(cb, DCH), :], gb_v)
        h0 = pltpu.async_copy(rows_v, xs_hbm.at[pos0_v.at[c]], sem)
        h1 = pltpu.async_copy(rows_v, xs_hbm.at[pos1_v.at[c]], sem)
        h2 = pltpu.async_copy(ga_v, gp_hbm.at[pos0_v.at[c]], sem)
        h3 = pltpu.async_copy(gb_v, gp_hbm.at[pos1_v.at[c]], sem)
        h0.wait()
        h1.wait()
        h2.wait()
        h3.wait()


def _dispatch(x, p0, p1, g0, g1, interpret=False):
    mesh = plsc.VectorSubcoreMesh(core_axis_name="c", subcore_axis_name="s")
    fn = pl.kernel(
        _dispatch_body,
        out_type=(
            jax.ShapeDtypeStruct((P, DI), jnp.float32),
            jax.ShapeDtypeStruct((P, 128), jnp.float32),
        ),
        mesh=mesh,
        scratch_types=[
            pltpu.VMEM((TPW // DCH, DCH), jnp.int32),
            pltpu.VMEM((TPW // DCH, DCH), jnp.int32),
            pltpu.VMEM((DCH, DI), jnp.float32),
            pltpu.VMEM((DCH, 128), jnp.float32),
            pltpu.VMEM((DCH, 128), jnp.float32),
            pltpu.SemaphoreType.DMA,
        ],
        interpret=interpret,
    )
    return fn(x, p0, p1, g0, g1)


def _gmm_body(be_ref, xs_ref, w_ref, b_ref, gp_ref, ys_ref):
    acc = jnp.dot(xs_ref[...], w_ref[0], preferred_element_type=jnp.float32)
    ys_ref[...] = (acc + b_ref[0]) * gp_ref[:, 0:1]


def _gmm(be, xs, gp, expert_w, expert_b, interpret=False):
    grid_spec = pltpu.PrefetchScalarGridSpec(
        num_scalar_prefetch=1,
        grid=(NBLK,),
        in_specs=[
            pl.BlockSpec((BM, DI), lambda i, be: (i, 0)),
            pl.BlockSpec((1, DI, DH), lambda i, be: (be[i], 0, 0)),
            pl.BlockSpec((1, 1, DH), lambda i, be: (be[i], 0, 0)),
            pl.BlockSpec((BM, 128), lambda i, be: (i, 0)),
        ],
        out_specs=pl.BlockSpec((BM, DH), lambda i, be: (i, 0)),
    )
    return pl.pallas_call(
        _gmm_body,
        grid_spec=grid_spec,
        out_shape=jax.ShapeDtypeStruct((P, DH), jnp.float32),
        interpret=interpret,
    )(be, xs, expert_w, expert_b.reshape(NE, 1, DH), gp)


def _combine_body(ys_hbm, pos0_hbm, pos1_hbm, out_hbm,
                  pos_v, buf_a, buf_b, out_v, sem_a, sem_b):
    w = lax.axis_index("s") * 2 + lax.axis_index("c")
    base = w * TPW
    pltpu.sync_copy(pos0_hbm.at[pl.ds(base, TPW)], pos_v.at[0])
    pltpu.sync_copy(pos1_hbm.at[pl.ds(base, TPW)], pos_v.at[1])

    def chunk(c, _):
        ha = pltpu.async_copy(ys_hbm.at[pos_v.at[0, pl.ds(c * CCH, CCH)]],
                              buf_a, sem_a)
        hb = pltpu.async_copy(ys_hbm.at[pos_v.at[1, pl.ds(c * CCH, CCH)]],
                              buf_b, sem_b)
        ha.wait()
        hb.wait()

        def tok(t, _):
            def vec(v, _):
                for u in range(4):
                    sl = pl.ds(v * 64 + u * 16, 16)
                    out_v[t, sl] = buf_a[t, sl] + buf_b[t, sl]
                return 0

            return lax.fori_loop(0, DH // 64, vec, 0)

        lax.fori_loop(0, CCH, tok, 0)
        pltpu.sync_copy(out_v, out_hbm.at[pl.ds(base + c * CCH, CCH)])
        return 0

    lax.fori_loop(0, TPW // CCH, chunk, 0)


def _combine(ys, pos0, pos1, interpret=False):
    mesh = plsc.VectorSubcoreMesh(core_axis_name="c", subcore_axis_name="s")
    fn = pl.kernel(
        _combine_body,
        out_type=jax.ShapeDtypeStruct((N, DH), jnp.float32),
        mesh=mesh,
        scratch_types=[
            pltpu.VMEM((2, TPW), jnp.int32),
            pltpu.VMEM((CCH, DH), jnp.float32),
            pltpu.VMEM((CCH, DH), jnp.float32),
            pltpu.VMEM((CCH, DH), jnp.float32),
            pltpu.SemaphoreType.DMA,
            pltpu.SemaphoreType.DMA,
        ],
        interpret=interpret,
    )
    return fn(ys, pos0, pos1)


def _pipeline(x, w_gate, expert_w, expert_b, interpret=False):
    e0, e1, r0, r1, g0, g1, counts = _routing(x, w_gate, interpret=interpret)
    c = counts[0]
    pc = ((c + BM - 1) // BM) * BM
    ends = jnp.cumsum(pc)
    poff = (ends - pc).astype(jnp.int32)
    starts = jnp.arange(NBLK, dtype=jnp.int32) * BM
    owns = ((starts[:, None] >= poff[None, :]) &
            (starts[:, None] < ends[None, :].astype(jnp.int32)))
    be = jnp.sum(jnp.where(owns, jnp.arange(NE, dtype=jnp.int32)[None, :], 0),
                 axis=1).astype(jnp.int32)
    p0, p1, g0r, g1r = _posmap(e0, e1, r0, r1, g0, g1, poff.reshape(1, NE),
                               interpret=interpret)
    p0, p1 = p0.reshape(N), p1.reshape(N)
    xs, gp = _dispatch(x, p0, p1, g0r, g1r, interpret=interpret)
    ys = _gmm(be, xs, gp, expert_w, expert_b, interpret=interpret)
    return _combine(ys, p0, p1, interpret=interpret)


def kernel(x, w_gate, w_noise, expert_w, expert_b):
    return _pipeline(x, w_gate, expert_w, expert_b)


# speedup vs baseline: 4.5157x; 1.2796x over previous
import functools

import jax
import jax.numpy as jnp
from jax import lax
from jax.experimental import pallas as pl
from jax.experimental.pallas import tpu as pltpu
from jax.experimental.pallas import tpu_sc as plsc

NE = 16
DI = 1024
DH = 4096
N = 8192

TB = 512
BM = 256
P = N * 2 + NE * BM
NBLK = P // BM
NW = 32
TPW = N // NW
DCH = 64
CCH = 4


def _routing_body(x_ref, wg_ref, e0_ref, e1_ref, r0_ref, r1_ref,
                  g0_ref, g1_ref, cnt_ref, carry):
    @pl.when(pl.program_id(0) == 0)
    def _():
        carry[...] = jnp.zeros_like(carry)

    logits = jnp.dot(x_ref[...], wg_ref[...],
                     preferred_element_type=jnp.float32)
    col = lax.broadcasted_iota(jnp.int32, (TB, NE), 1)
    m0 = jnp.max(logits, axis=1, keepdims=True)
    i0 = jnp.min(jnp.where(logits == m0, col, NE), axis=1, keepdims=True)
    masked = jnp.where(col == i0, -jnp.inf, logits)
    m1 = jnp.max(masked, axis=1, keepdims=True)
    i1 = jnp.min(jnp.where(masked == m1, col, NE), axis=1, keepdims=True)
    e1x = jnp.exp(m1 - m0)
    den = 1.0 + e1x
    g0 = 1.0 / den
    g1 = e1x / den

    oh0 = (col == i0).astype(jnp.float32)
    oh1 = (col == i1).astype(jnp.float32)
    oh = oh0 + oh1
    r_io = lax.broadcasted_iota(jnp.int32, (TB, TB), 0)
    c_io = lax.broadcasted_iota(jnp.int32, (TB, TB), 1)
    tri = (c_io < r_io).astype(jnp.float32)
    csum = jnp.dot(tri, oh, preferred_element_type=jnp.float32)
    pre = csum + carry[...]
    r0 = jnp.sum(oh0 * pre, axis=1, keepdims=True)
    r1 = jnp.sum(oh1 * pre, axis=1, keepdims=True)
    new_carry = carry[...] + jnp.sum(oh, axis=0, keepdims=True)
    carry[...] = new_carry

    e0_ref[...] = i0
    e1_ref[...] = i1
    r0_ref[...] = r0.astype(jnp.int32)
    r1_ref[...] = r1.astype(jnp.int32)
    g0_ref[...] = g0
    g1_ref[...] = g1
    cnt_ref[...] = new_carry.astype(jnp.int32)


def _routing(x, w_gate, interpret=False):
    nblk = N // TB
    out_shapes = (
        jax.ShapeDtypeStruct((N, 1), jnp.int32),
        jax.ShapeDtypeStruct((N, 1), jnp.int32),
        jax.ShapeDtypeStruct((N, 1), jnp.int32),
        jax.ShapeDtypeStruct((N, 1), jnp.int32),
        jax.ShapeDtypeStruct((N, 1), jnp.float32),
        jax.ShapeDtypeStruct((N, 1), jnp.float32),
        jax.ShapeDtypeStruct((1, NE), jnp.int32),
    )
    tok_spec = pl.BlockSpec((TB, 1), lambda i: (i, 0))
    return pl.pallas_call(
        _routing_body,
        grid=(nblk,),
        in_specs=[
            pl.BlockSpec((TB, DI), lambda i: (i, 0)),
            pl.BlockSpec((DI, NE), lambda i: (0, 0)),
        ],
        out_specs=(tok_spec, tok_spec, tok_spec, tok_spec, tok_spec, tok_spec,
                   pl.BlockSpec((1, NE), lambda i: (0, 0))),
        out_shape=out_shapes,
        scratch_shapes=[pltpu.VMEM((1, NE), jnp.float32)],
        interpret=interpret,
    )(x, w_gate)


def _posmap_body(e0_ref, e1_ref, r0_ref, r1_ref, g0_ref, g1_ref, poff_ref,
                 p0_ref, p1_ref, g0r_ref, g1r_ref):
    col = lax.broadcasted_iota(jnp.int32, (TB, NE), 1)
    poff_f = poff_ref[...].astype(jnp.float32)
    oh0 = (col == e0_ref[...]).astype(jnp.float32)
    oh1 = (col == e1_ref[...]).astype(jnp.float32)
    p0 = r0_ref[...].astype(jnp.float32) + jnp.sum(oh0 * poff_f, axis=1,
                                                   keepdims=True)
    p1 = r1_ref[...].astype(jnp.float32) + jnp.sum(oh1 * poff_f, axis=1,
                                                   keepdims=True)
    p0_ref[...] = p0.astype(jnp.int32)
    p1_ref[...] = p1.astype(jnp.int32)
    g0r_ref[...] = jnp.broadcast_to(g0_ref[...], (TB, 128))
    g1r_ref[...] = jnp.broadcast_to(g1_ref[...], (TB, 128))


def _posmap(e0, e1, r0, r1, g0, g1, poff, interpret=False):
    tok_spec = pl.BlockSpec((TB, 1), lambda i: (i, 0))
    rep_spec = pl.BlockSpec((TB, 128), lambda i: (i, 0))
    return pl.pallas_call(
        _posmap_body,
        grid=(N // TB,),
        in_specs=[tok_spec, tok_spec, tok_spec, tok_spec, tok_spec, tok_spec,
                  pl.BlockSpec((1, NE), lambda i: (0, 0))],
        out_specs=(tok_spec, tok_spec, rep_spec, rep_spec),
        out_shape=(jax.ShapeDtypeStruct((N, 1), jnp.int32),
                   jax.ShapeDtypeStruct((N, 1), jnp.int32),
                   jax.ShapeDtypeStruct((N, 128), jnp.float32),
                   jax.ShapeDtypeStruct((N, 128), jnp.float32)),
        interpret=interpret,
    )(e0, e1, r0, r1, g0, g1, poff)


def _dispatch_body(x_hbm, p0_hbm, p1_hbm, g0_hbm, g1_hbm,
                   xs_hbm, gp_hbm,
                   pos0_v, pos1_v, rows_v, ga_v, gb_v, sem):
    w = lax.axis_index("s") * 2 + lax.axis_index("c")
    base = w * TPW
    nch = TPW // DCH
    for c in range(nch):
        cb = base + c * DCH
        pltpu.sync_copy(p0_hbm.at[pl.ds(cb, DCH)], pos0_v.at[c])
        pltpu.sync_copy(p1_hbm.at[pl.ds(cb, DCH)], pos1_v.at[c])
    for c in range(nch):
        cb = base + c * DCH
        pltpu.sync_copy(x_hbm.at[pl.ds(cb, DCH)], rows_v)
        pltpu.sync_copy(g0_hbm.at[pl.ds(cb, DCH), :], ga_v)
        pltpu.sync_copy(g1_hbm.at[pl.ds(cb, DCH), :], gb_v)
        h0 = pltpu.async_copy(rows_v, xs_hbm.at[pos0_v.at[c]], sem)
        h1 = pltpu.async_copy(rows_v, xs_hbm.at[pos1_v.at[c]], sem)
        h2 = pltpu.async_copy(ga_v, gp_hbm.at[pos0_v.at[c]], sem)
        h3 = pltpu.async_copy(gb_v, gp_hbm.at[pos1_v.at[c]], sem)
        h0.wait()
        h1.wait()
        h2.wait()
        h3.wait()


def _dispatch(x, p0, p1, g0, g1, interpret=False):
    mesh = plsc.VectorSubcoreMesh(core_axis_name="c", subcore_axis_name="s")
    fn = pl.kernel(
        _dispatch_body,
        out_type=(
            jax.ShapeDtypeStruct((P, DI), jnp.float32),
            jax.ShapeDtypeStruct((P, 128), jnp.float32),
        ),
        mesh=mesh,
        scratch_types=[
            pltpu.VMEM((TPW // DCH, DCH), jnp.int32),
            pltpu.VMEM((TPW // DCH, DCH), jnp.int32),
            pltpu.VMEM((DCH, DI), jnp.float32),
            pltpu.VMEM((DCH, 128), jnp.float32),
            pltpu.VMEM((DCH, 128), jnp.float32),
            pltpu.SemaphoreType.DMA,
        ],
        interpret=interpret,
    )
    return fn(x, p0, p1, g0, g1)


def _gmm_body(be_ref, xs_ref, w_ref, b_ref, gp_ref, ys_ref):
    acc = jnp.dot(xs_ref[...], w_ref[0], preferred_element_type=jnp.float32)
    ys_ref[...] = (acc + b_ref[0]) * gp_ref[:, 0:1]


def _gmm(be, xs, gp, expert_w, expert_b, interpret=False):
    grid_spec = pltpu.PrefetchScalarGridSpec(
        num_scalar_prefetch=1,
        grid=(NBLK,),
        in_specs=[
            pl.BlockSpec((BM, DI), lambda i, be: (i, 0)),
            pl.BlockSpec((1, DI, DH), lambda i, be: (be[i], 0, 0)),
            pl.BlockSpec((1, 1, DH), lambda i, be: (be[i], 0, 0)),
            pl.BlockSpec((BM, 128), lambda i, be: (i, 0)),
        ],
        out_specs=pl.BlockSpec((BM, DH), lambda i, be: (i, 0)),
    )
    return pl.pallas_call(
        _gmm_body,
        grid_spec=grid_spec,
        out_shape=jax.ShapeDtypeStruct((P, DH), jnp.float32),
        interpret=interpret,
    )(be, xs, expert_w, expert_b.reshape(NE, 1, DH), gp)


def _combine_body(ys_hbm, pos0_hbm, pos1_hbm, out_hbm,
                  pos_v, a0, a1, b0, b1, g0s, g1s, w0s, w1s):
    w = lax.axis_index("s") * 2 + lax.axis_index("c")
    base = w * TPW
    pltpu.sync_copy(pos0_hbm.at[pl.ds(base, TPW)], pos_v.at[0])
    pltpu.sync_copy(pos1_hbm.at[pl.ds(base, TPW)], pos_v.at[1])

    abufs = (a0, a1)
    bbufs = (b0, b1)
    gsems = (g0s, g1s)
    wsems = (w0s, w1s)

    def pair(g, _):
        c0 = 2 * g
        gh = []
        for b in range(2):
            c = c0 + b
            gh.append(pltpu.async_copy(
                ys_hbm.at[pos_v.at[0, pl.ds(c * CCH, CCH)]],
                abufs[b], gsems[b]))
            gh.append(pltpu.async_copy(
                ys_hbm.at[pos_v.at[1, pl.ds(c * CCH, CCH)]],
                bbufs[b], gsems[b]))
        wh = []
        for b in range(2):
            c = c0 + b
            gh[2 * b].wait()
            gh[2 * b + 1].wait()
            for t in range(CCH):
                def vec(v, _, t=t, b=b):
                    for u in range(4):
                        sl = pl.ds(v * 64 + u * 16, 16)
                        plsc.addupdate(abufs[b].at[t, sl], bbufs[b][t, sl])
                    return 0

                lax.fori_loop(0, DH // 64, vec, 0)
            wh.append(pltpu.async_copy(
                abufs[b], out_hbm.at[pl.ds(base + c * CCH, CCH)], wsems[b]))
        wh[0].wait()
        wh[1].wait()
        return 0

    lax.fori_loop(0, (TPW // CCH) // 2, pair, 0)


def _combine(ys, pos0, pos1, interpret=False):
    mesh = plsc.VectorSubcoreMesh(core_axis_name="c", subcore_axis_name="s")
    bufty = pltpu.VMEM((CCH, DH), jnp.float32)
    fn = pl.kernel(
        _combine_body,
        out_type=jax.ShapeDtypeStruct((N, DH), jnp.float32),
        mesh=mesh,
        scratch_types=[
            pltpu.VMEM((2, TPW), jnp.int32),
            bufty, bufty, bufty, bufty,
            pltpu.SemaphoreType.DMA,
            pltpu.SemaphoreType.DMA,
            pltpu.SemaphoreType.DMA,
            pltpu.SemaphoreType.DMA,
        ],
        interpret=interpret,
    )
    return fn(ys, pos0, pos1)


def _pipeline(x, w_gate, expert_w, expert_b, interpret=False):
    e0, e1, r0, r1, g0, g1, counts = _routing(x, w_gate, interpret=interpret)
    c = counts[0]
    pc = ((c + BM - 1) // BM) * BM
    ends = jnp.cumsum(pc)
    poff = (ends - pc).astype(jnp.int32)
    starts = jnp.arange(NBLK, dtype=jnp.int32) * BM
    owns = ((starts[:, None] >= poff[None, :]) &
            (starts[:, None] < ends[None, :].astype(jnp.int32)))
    be = jnp.sum(jnp.where(owns, jnp.arange(NE, dtype=jnp.int32)[None, :], 0),
                 axis=1).astype(jnp.int32)
    p0, p1, g0r, g1r = _posmap(e0, e1, r0, r1, g0, g1, poff.reshape(1, NE),
                               interpret=interpret)
    p0, p1 = p0.reshape(N), p1.reshape(N)
    xs, gp = _dispatch(x, p0, p1, g0r, g1r, interpret=interpret)
    ys = _gmm(be, xs, gp, expert_w, expert_b, interpret=interpret)
    return _combine(ys, p0, p1, interpret=interpret)


def kernel(x, w_gate, w_noise, expert_w, expert_b):
    return _pipeline(x, w_gate, expert_w, expert_b)


# speedup vs baseline: 4.5215x; 1.0013x over previous
import functools

import jax
import jax.numpy as jnp
from jax import lax
from jax.experimental import pallas as pl
from jax.experimental.pallas import tpu as pltpu
from jax.experimental.pallas import tpu_sc as plsc

NE = 16
DI = 1024
DH = 4096
N = 8192

TB = 512
BM = 256
P = N * 2 + NE * BM
NBLK = P // BM
NW = 32
TPW = N // NW
DCH = 64
CCH = 4


def _routing_body(x_ref, wg_ref, e0_ref, e1_ref, r0_ref, r1_ref,
                  g0_ref, g1_ref, cnt_ref, carry):
    @pl.when(pl.program_id(0) == 0)
    def _():
        carry[...] = jnp.zeros_like(carry)

    logits = jnp.dot(x_ref[...], wg_ref[...],
                     preferred_element_type=jnp.float32)
    col = lax.broadcasted_iota(jnp.int32, (TB, NE), 1)
    m0 = jnp.max(logits, axis=1, keepdims=True)
    i0 = jnp.min(jnp.where(logits == m0, col, NE), axis=1, keepdims=True)
    masked = jnp.where(col == i0, -jnp.inf, logits)
    m1 = jnp.max(masked, axis=1, keepdims=True)
    i1 = jnp.min(jnp.where(masked == m1, col, NE), axis=1, keepdims=True)
    e1x = jnp.exp(m1 - m0)
    den = 1.0 + e1x
    g0 = 1.0 / den
    g1 = e1x / den

    oh0 = (col == i0).astype(jnp.float32)
    oh1 = (col == i1).astype(jnp.float32)
    oh = oh0 + oh1
    r_io = lax.broadcasted_iota(jnp.int32, (TB, TB), 0)
    c_io = lax.broadcasted_iota(jnp.int32, (TB, TB), 1)
    tri = (c_io < r_io).astype(jnp.float32)
    csum = jnp.dot(tri, oh, preferred_element_type=jnp.float32)
    pre = csum + carry[...]
    r0 = jnp.sum(oh0 * pre, axis=1, keepdims=True)
    r1 = jnp.sum(oh1 * pre, axis=1, keepdims=True)
    new_carry = carry[...] + jnp.sum(oh, axis=0, keepdims=True)
    carry[...] = new_carry

    e0_ref[...] = i0
    e1_ref[...] = i1
    r0_ref[...] = r0.astype(jnp.int32)
    r1_ref[...] = r1.astype(jnp.int32)
    g0_ref[...] = g0
    g1_ref[...] = g1
    cnt_ref[...] = new_carry.astype(jnp.int32)


def _routing(x, w_gate, interpret=False):
    nblk = N // TB
    out_shapes = (
        jax.ShapeDtypeStruct((N, 1), jnp.int32),
        jax.ShapeDtypeStruct((N, 1), jnp.int32),
        jax.ShapeDtypeStruct((N, 1), jnp.int32),
        jax.ShapeDtypeStruct((N, 1), jnp.int32),
        jax.ShapeDtypeStruct((N, 1), jnp.float32),
        jax.ShapeDtypeStruct((N, 1), jnp.float32),
        jax.ShapeDtypeStruct((1, NE), jnp.int32),
    )
    tok_spec = pl.BlockSpec((TB, 1), lambda i: (i, 0))
    return pl.pallas_call(
        _routing_body,
        grid=(nblk,),
        in_specs=[
            pl.BlockSpec((TB, DI), lambda i: (i, 0)),
            pl.BlockSpec((DI, NE), lambda i: (0, 0)),
        ],
        out_specs=(tok_spec, tok_spec, tok_spec, tok_spec, tok_spec, tok_spec,
                   pl.BlockSpec((1, NE), lambda i: (0, 0))),
        out_shape=out_shapes,
        scratch_shapes=[pltpu.VMEM((1, NE), jnp.float32)],
        interpret=interpret,
    )(x, w_gate)


def _posmap_body(e0_ref, e1_ref, r0_ref, r1_ref, g0_ref, g1_ref, poff_ref,
                 p0_ref, p1_ref, g0r_ref, g1r_ref):
    col = lax.broadcasted_iota(jnp.int32, (TB, NE), 1)
    poff_f = poff_ref[...].astype(jnp.float32)
    oh0 = (col == e0_ref[...]).astype(jnp.float32)
    oh1 = (col == e1_ref[...]).astype(jnp.float32)
    p0 = r0_ref[...].astype(jnp.float32) + jnp.sum(oh0 * poff_f, axis=1,
                                                   keepdims=True)
    p1 = r1_ref[...].astype(jnp.float32) + jnp.sum(oh1 * poff_f, axis=1,
                                                   keepdims=True)
    p0_ref[...] = p0.astype(jnp.int32)
    p1_ref[...] = p1.astype(jnp.int32)
    g0r_ref[...] = jnp.broadcast_to(g0_ref[...], (TB, 128))
    g1r_ref[...] = jnp.broadcast_to(g1_ref[...], (TB, 128))


def _posmap(e0, e1, r0, r1, g0, g1, poff, interpret=False):
    tok_spec = pl.BlockSpec((TB, 1), lambda i: (i, 0))
    rep_spec = pl.BlockSpec((TB, 128), lambda i: (i, 0))
    return pl.pallas_call(
        _posmap_body,
        grid=(N // TB,),
        in_specs=[tok_spec, tok_spec, tok_spec, tok_spec, tok_spec, tok_spec,
                  pl.BlockSpec((1, NE), lambda i: (0, 0))],
        out_specs=(tok_spec, tok_spec, rep_spec, rep_spec),
        out_shape=(jax.ShapeDtypeStruct((N, 1), jnp.int32),
                   jax.ShapeDtypeStruct((N, 1), jnp.int32),
                   jax.ShapeDtypeStruct((N, 128), jnp.float32),
                   jax.ShapeDtypeStruct((N, 128), jnp.float32)),
        interpret=interpret,
    )(e0, e1, r0, r1, g0, g1, poff)


def _dispatch_body(x_hbm, p0_hbm, p1_hbm, g0_hbm, g1_hbm,
                   xs_hbm, gp_hbm,
                   pos0_v, pos1_v, rows_v, ga_v, gb_v, sem):
    w = lax.axis_index("s") * 2 + lax.axis_index("c")
    base = w * TPW
    nch = TPW // DCH
    for c in range(nch):
        cb = base + c * DCH
        pltpu.sync_copy(p0_hbm.at[pl.ds(cb, DCH)], pos0_v.at[c])
        pltpu.sync_copy(p1_hbm.at[pl.ds(cb, DCH)], pos1_v.at[c])
    for c in range(nch):
        cb = base + c * DCH
        pltpu.sync_copy(x_hbm.at[pl.ds(cb, DCH)], rows_v)
        pltpu.sync_copy(g0_hbm.at[pl.ds(cb, DCH), :], ga_v)
        pltpu.sync_copy(g1_hbm.at[pl.ds(cb, DCH), :], gb_v)
        h0 = pltpu.async_copy(rows_v, xs_hbm.at[pos0_v.at[c]], sem)
        h1 = pltpu.async_copy(rows_v, xs_hbm.at[pos1_v.at[c]], sem)
        h2 = pltpu.async_copy(ga_v, gp_hbm.at[pos0_v.at[c]], sem)
        h3 = pltpu.async_copy(gb_v, gp_hbm.at[pos1_v.at[c]], sem)
        h0.wait()
        h1.wait()
        h2.wait()
        h3.wait()


def _dispatch(x, p0, p1, g0, g1, interpret=False):
    mesh = plsc.VectorSubcoreMesh(core_axis_name="c", subcore_axis_name="s")
    fn = pl.kernel(
        _dispatch_body,
        out_type=(
            jax.ShapeDtypeStruct((P, DI), jnp.float32),
            jax.ShapeDtypeStruct((P, 128), jnp.float32),
        ),
        mesh=mesh,
        scratch_types=[
            pltpu.VMEM((TPW // DCH, DCH), jnp.int32),
            pltpu.VMEM((TPW // DCH, DCH), jnp.int32),
            pltpu.VMEM((DCH, DI), jnp.float32),
            pltpu.VMEM((DCH, 128), jnp.float32),
            pltpu.VMEM((DCH, 128), jnp.float32),
            pltpu.SemaphoreType.DMA,
        ],
        interpret=interpret,
    )
    return fn(x, p0, p1, g0, g1)


def _gmm_body(be_ref, xs_ref, w_ref, b_ref, gp_ref, ys_ref):
    acc = jnp.dot(xs_ref[...].astype(jnp.bfloat16),
                  w_ref[0].astype(jnp.bfloat16),
                  preferred_element_type=jnp.float32)
    ys_ref[...] = (acc + b_ref[0]) * gp_ref[:, 0:1]


def _gmm(be, xs, gp, expert_w, expert_b, interpret=False):
    grid_spec = pltpu.PrefetchScalarGridSpec(
        num_scalar_prefetch=1,
        grid=(NBLK,),
        in_specs=[
            pl.BlockSpec((BM, DI), lambda i, be: (i, 0)),
            pl.BlockSpec((1, DI, DH), lambda i, be: (be[i], 0, 0)),
            pl.BlockSpec((1, 1, DH), lambda i, be: (be[i], 0, 0)),
            pl.BlockSpec((BM, 128), lambda i, be: (i, 0)),
        ],
        out_specs=pl.BlockSpec((BM, DH), lambda i, be: (i, 0)),
    )
    return pl.pallas_call(
        _gmm_body,
        grid_spec=grid_spec,
        out_shape=jax.ShapeDtypeStruct((P, DH), jnp.float32),
        interpret=interpret,
    )(be, xs, expert_w, expert_b.reshape(NE, 1, DH), gp)


def _combine_body(ys_hbm, pos0_hbm, pos1_hbm, out_hbm,
                  pos_v, a0, a1, b0, b1, g0s, g1s, w0s, w1s):
    w = lax.axis_index("s") * 2 + lax.axis_index("c")
    base = w * TPW
    pltpu.sync_copy(pos0_hbm.at[pl.ds(base, TPW)], pos_v.at[0])
    pltpu.sync_copy(pos1_hbm.at[pl.ds(base, TPW)], pos_v.at[1])

    abufs = (a0, a1)
    bbufs = (b0, b1)
    gsems = (g0s, g1s)
    wsems = (w0s, w1s)

    def pair(g, _):
        c0 = 2 * g
        gh = []
        for b in range(2):
            c = c0 + b
            gh.append(pltpu.async_copy(
                ys_hbm.at[pos_v.at[0, pl.ds(c * CCH, CCH)]],
                abufs[b], gsems[b]))
            gh.append(pltpu.async_copy(
                ys_hbm.at[pos_v.at[1, pl.ds(c * CCH, CCH)]],
                bbufs[b], gsems[b]))
        wh = []
        for b in range(2):
            c = c0 + b
            gh[2 * b].wait()
            gh[2 * b + 1].wait()
            for t in range(CCH):
                def vec(v, _, t=t, b=b):
                    for u in range(4):
                        sl = pl.ds(v * 64 + u * 16, 16)
                        plsc.addupdate(abufs[b].at[t, sl], bbufs[b][t, sl])
                    return 0

                lax.fori_loop(0, DH // 64, vec, 0)
            wh.append(pltpu.async_copy(
                abufs[b], out_hbm.at[pl.ds(base + c * CCH, CCH)], wsems[b]))
        wh[0].wait()
        wh[1].wait()
        return 0

    lax.fori_loop(0, (TPW // CCH) // 2, pair, 0)


def _combine(ys, pos0, pos1, interpret=False):
    mesh = plsc.VectorSubcoreMesh(core_axis_name="c", subcore_axis_name="s")
    bufty = pltpu.VMEM((CCH, DH), jnp.float32)
    fn = pl.kernel(
        _combine_body,
        out_type=jax.ShapeDtypeStruct((N, DH), jnp.float32),
        mesh=mesh,
        scratch_types=[
            pltpu.VMEM((2, TPW), jnp.int32),
            bufty, bufty, bufty, bufty,
            pltpu.SemaphoreType.DMA,
            pltpu.SemaphoreType.DMA,
            pltpu.SemaphoreType.DMA,
            pltpu.SemaphoreType.DMA,
        ],
        interpret=interpret,
    )
    return fn(ys, pos0, pos1)


def _pipeline(x, w_gate, expert_w, expert_b, interpret=False):
    e0, e1, r0, r1, g0, g1, counts = _routing(x, w_gate, interpret=interpret)
    c = counts[0]
    pc = ((c + BM - 1) // BM) * BM
    ends = jnp.cumsum(pc)
    poff = (ends - pc).astype(jnp.int32)
    starts = jnp.arange(NBLK, dtype=jnp.int32) * BM
    owns = ((starts[:, None] >= poff[None, :]) &
            (starts[:, None] < ends[None, :].astype(jnp.int32)))
    be = jnp.sum(jnp.where(owns, jnp.arange(NE, dtype=jnp.int32)[None, :], 0),
                 axis=1).astype(jnp.int32)
    p0, p1, g0r, g1r = _posmap(e0, e1, r0, r1, g0, g1, poff.reshape(1, NE),
                               interpret=interpret)
    p0, p1 = p0.reshape(N), p1.reshape(N)
    xs, gp = _dispatch(x, p0, p1, g0r, g1r, interpret=interpret)
    ys = _gmm(be, xs, gp, expert_w, expert_b, interpret=interpret)
    return _combine(ys, p0, p1, interpret=interpret)


def kernel(x, w_gate, w_noise, expert_w, expert_b):
    return _pipeline(x, w_gate, expert_w, expert_b)


# speedup vs baseline: 4.5394x; 1.0040x over previous
import functools

import jax
import jax.numpy as jnp
from jax import lax
from jax.experimental import pallas as pl
from jax.experimental.pallas import tpu as pltpu
from jax.experimental.pallas import tpu_sc as plsc

NE = 16
DI = 1024
DH = 4096
N = 8192

TB = 512
BM = 256
P = N * 2 + NE * BM
NBLK = P // BM
NW = 32
TPW = N // NW
DCH = 64
CCH = 4


def _routing_body(x_ref, wg_ref, e0_ref, e1_ref, r0_ref, r1_ref,
                  g0_ref, g1_ref, poff_ref, be_ref, carry):
    @pl.when(pl.program_id(0) == 0)
    def _():
        carry[...] = jnp.zeros_like(carry)

    logits = jnp.dot(x_ref[...], wg_ref[...],
                     preferred_element_type=jnp.float32)
    col = lax.broadcasted_iota(jnp.int32, (TB, NE), 1)
    m0 = jnp.max(logits, axis=1, keepdims=True)
    i0 = jnp.min(jnp.where(logits == m0, col, NE), axis=1, keepdims=True)
    masked = jnp.where(col == i0, -jnp.inf, logits)
    m1 = jnp.max(masked, axis=1, keepdims=True)
    i1 = jnp.min(jnp.where(masked == m1, col, NE), axis=1, keepdims=True)
    e1x = jnp.exp(m1 - m0)
    den = 1.0 + e1x
    g0 = 1.0 / den
    g1 = e1x / den

    oh0 = (col == i0).astype(jnp.float32)
    oh1 = (col == i1).astype(jnp.float32)
    oh = oh0 + oh1
    r_io = lax.broadcasted_iota(jnp.int32, (TB, TB), 0)
    c_io = lax.broadcasted_iota(jnp.int32, (TB, TB), 1)
    tri = (c_io < r_io).astype(jnp.float32)
    csum = jnp.dot(tri, oh, preferred_element_type=jnp.float32)
    pre = csum + carry[...]
    r0 = jnp.sum(oh0 * pre, axis=1, keepdims=True)
    r1 = jnp.sum(oh1 * pre, axis=1, keepdims=True)
    new_carry = carry[...] + jnp.sum(oh, axis=0, keepdims=True)
    carry[...] = new_carry

    e0_ref[...] = i0
    e1_ref[...] = i1
    r0_ref[...] = r0.astype(jnp.int32)
    r1_ref[...] = r1.astype(jnp.int32)
    g0_ref[...] = g0
    g1_ref[...] = g1
    pc = jnp.floor((new_carry + (BM - 1)) * (1.0 / BM)) * BM
    eio0 = lax.broadcasted_iota(jnp.int32, (NE, NE), 0)
    eio1 = lax.broadcasted_iota(jnp.int32, (NE, NE), 1)
    incl = (eio0 <= eio1).astype(jnp.float32)
    ends = jnp.dot(pc, incl, preferred_element_type=jnp.float32)
    poff = ends - pc
    poff_ref[...] = poff.astype(jnp.int32)
    bio0 = (lax.broadcasted_iota(jnp.int32, (128, NE), 0)
            .astype(jnp.float32) * BM)
    bio1 = lax.broadcasted_iota(jnp.int32, (128, NE), 1)
    owns = (bio0 >= poff) & (bio0 < ends)
    be_ref[...] = jnp.sum(jnp.where(owns, bio1, 0), axis=1,
                          keepdims=True).astype(jnp.int32)


def _routing(x, w_gate, interpret=False):
    nblk = N // TB
    out_shapes = (
        jax.ShapeDtypeStruct((N, 1), jnp.int32),
        jax.ShapeDtypeStruct((N, 1), jnp.int32),
        jax.ShapeDtypeStruct((N, 1), jnp.int32),
        jax.ShapeDtypeStruct((N, 1), jnp.int32),
        jax.ShapeDtypeStruct((N, 1), jnp.float32),
        jax.ShapeDtypeStruct((N, 1), jnp.float32),
        jax.ShapeDtypeStruct((1, NE), jnp.int32),
        jax.ShapeDtypeStruct((128, 1), jnp.int32),
    )
    tok_spec = pl.BlockSpec((TB, 1), lambda i: (i, 0))
    return pl.pallas_call(
        _routing_body,
        grid=(nblk,),
        in_specs=[
            pl.BlockSpec((TB, DI), lambda i: (i, 0)),
            pl.BlockSpec((DI, NE), lambda i: (0, 0)),
        ],
        out_specs=(tok_spec, tok_spec, tok_spec, tok_spec, tok_spec, tok_spec,
                   pl.BlockSpec((1, NE), lambda i: (0, 0)),
                   pl.BlockSpec((128, 1), lambda i: (0, 0))),
        out_shape=out_shapes,
        scratch_shapes=[pltpu.VMEM((1, NE), jnp.float32)],
        interpret=interpret,
    )(x, w_gate)


def _posmap_body(e0_ref, e1_ref, r0_ref, r1_ref, g0_ref, g1_ref, poff_ref,
                 p0_ref, p1_ref, g0r_ref, g1r_ref):
    col = lax.broadcasted_iota(jnp.int32, (TB, NE), 1)
    poff_f = poff_ref[...].astype(jnp.float32)
    oh0 = (col == e0_ref[...]).astype(jnp.float32)
    oh1 = (col == e1_ref[...]).astype(jnp.float32)
    p0 = r0_ref[...].astype(jnp.float32) + jnp.sum(oh0 * poff_f, axis=1,
                                                   keepdims=True)
    p1 = r1_ref[...].astype(jnp.float32) + jnp.sum(oh1 * poff_f, axis=1,
                                                   keepdims=True)
    p0_ref[...] = p0.astype(jnp.int32)
    p1_ref[...] = p1.astype(jnp.int32)
    g0r_ref[...] = jnp.broadcast_to(g0_ref[...], (TB, 128))
    g1r_ref[...] = jnp.broadcast_to(g1_ref[...], (TB, 128))


def _posmap(e0, e1, r0, r1, g0, g1, poff, interpret=False):
    tok_spec = pl.BlockSpec((TB, 1), lambda i: (i, 0))
    rep_spec = pl.BlockSpec((TB, 128), lambda i: (i, 0))
    return pl.pallas_call(
        _posmap_body,
        grid=(N // TB,),
        in_specs=[tok_spec, tok_spec, tok_spec, tok_spec, tok_spec, tok_spec,
                  pl.BlockSpec((1, NE), lambda i: (0, 0))],
        out_specs=(tok_spec, tok_spec, rep_spec, rep_spec),
        out_shape=(jax.ShapeDtypeStruct((N, 1), jnp.int32),
                   jax.ShapeDtypeStruct((N, 1), jnp.int32),
                   jax.ShapeDtypeStruct((N, 128), jnp.float32),
                   jax.ShapeDtypeStruct((N, 128), jnp.float32)),
        interpret=interpret,
    )(e0, e1, r0, r1, g0, g1, poff)


def _dispatch_body(x_hbm, p0_hbm, p1_hbm, g0_hbm, g1_hbm,
                   xs_hbm, gp_hbm,
                   pos0_v, pos1_v, rows_v, ga_v, gb_v, sem):
    w = lax.axis_index("s") * 2 + lax.axis_index("c")
    base = w * TPW
    nch = TPW // DCH
    for c in range(nch):
        cb = base + c * DCH
        pltpu.sync_copy(p0_hbm.at[pl.ds(cb, DCH)], pos0_v.at[c])
        pltpu.sync_copy(p1_hbm.at[pl.ds(cb, DCH)], pos1_v.at[c])
    for c in range(nch):
        cb = base + c * DCH
        pltpu.sync_copy(x_hbm.at[pl.ds(cb, DCH)], rows_v)
        pltpu.sync_copy(g0_hbm.at[pl.ds(cb, DCH), :], ga_v)
        pltpu.sync_copy(g1_hbm.at[pl.ds(cb, DCH), :], gb_v)
        h0 = pltpu.async_copy(rows_v, xs_hbm.at[pos0_v.at[c]], sem)
        h1 = pltpu.async_copy(rows_v, xs_hbm.at[pos1_v.at[c]], sem)
        h2 = pltpu.async_copy(ga_v, gp_hbm.at[pos0_v.at[c]], sem)
        h3 = pltpu.async_copy(gb_v, gp_hbm.at[pos1_v.at[c]], sem)
        h0.wait()
        h1.wait()
        h2.wait()
        h3.wait()


def _dispatch(x, p0, p1, g0, g1, interpret=False):
    mesh = plsc.VectorSubcoreMesh(core_axis_name="c", subcore_axis_name="s")
    fn = pl.kernel(
        _dispatch_body,
        out_type=(
            jax.ShapeDtypeStruct((P, DI), jnp.float32),
            jax.ShapeDtypeStruct((P, 128), jnp.float32),
        ),
        mesh=mesh,
        scratch_types=[
            pltpu.VMEM((TPW // DCH, DCH), jnp.int32),
            pltpu.VMEM((TPW // DCH, DCH), jnp.int32),
            pltpu.VMEM((DCH, DI), jnp.float32),
            pltpu.VMEM((DCH, 128), jnp.float32),
            pltpu.VMEM((DCH, 128), jnp.float32),
            pltpu.SemaphoreType.DMA,
        ],
        interpret=interpret,
    )
    return fn(x, p0, p1, g0, g1)


def _gmm_body(be_ref, xs_ref, w_ref, b_ref, gp_ref, ys_ref):
    acc = jnp.dot(xs_ref[...].astype(jnp.bfloat16),
                  w_ref[0].astype(jnp.bfloat16),
                  preferred_element_type=jnp.float32)
    ys_ref[...] = (acc + b_ref[0]) * gp_ref[:, 0:1]


def _gmm(be, xs, gp, expert_w, expert_b, interpret=False):
    grid_spec = pltpu.PrefetchScalarGridSpec(
        num_scalar_prefetch=1,
        grid=(NBLK,),
        in_specs=[
            pl.BlockSpec((BM, DI), lambda i, be: (i, 0)),
            pl.BlockSpec((1, DI, DH), lambda i, be: (be[i], 0, 0)),
            pl.BlockSpec((1, 1, DH), lambda i, be: (be[i], 0, 0)),
            pl.BlockSpec((BM, 128), lambda i, be: (i, 0)),
        ],
        out_specs=pl.BlockSpec((BM, DH), lambda i, be: (i, 0)),
    )
    return pl.pallas_call(
        _gmm_body,
        grid_spec=grid_spec,
        out_shape=jax.ShapeDtypeStruct((P, DH), jnp.float32),
        interpret=interpret,
    )(be, xs, expert_w, expert_b.reshape(NE, 1, DH), gp)


def _combine_body(ys_hbm, pos0_hbm, pos1_hbm, out_hbm,
                  pos_v, a0, a1, b0, b1, g0s, g1s, w0s, w1s):
    w = lax.axis_index("s") * 2 + lax.axis_index("c")
    base = w * TPW
    pltpu.sync_copy(pos0_hbm.at[pl.ds(base, TPW)], pos_v.at[0])
    pltpu.sync_copy(pos1_hbm.at[pl.ds(base, TPW)], pos_v.at[1])

    abufs = (a0, a1)
    bbufs = (b0, b1)
    gsems = (g0s, g1s)
    wsems = (w0s, w1s)

    def pair(g, _):
        c0 = 2 * g
        gh = []
        for b in range(2):
            c = c0 + b
            gh.append(pltpu.async_copy(
                ys_hbm.at[pos_v.at[0, pl.ds(c * CCH, CCH)]],
                abufs[b], gsems[b]))
            gh.append(pltpu.async_copy(
                ys_hbm.at[pos_v.at[1, pl.ds(c * CCH, CCH)]],
                bbufs[b], gsems[b]))
        wh = []
        for b in range(2):
            c = c0 + b
            gh[2 * b].wait()
            gh[2 * b + 1].wait()
            for t in range(CCH):
                def vec(v, _, t=t, b=b):
                    for u in range(4):
                        sl = pl.ds(v * 64 + u * 16, 16)
                        plsc.addupdate(abufs[b].at[t, sl], bbufs[b][t, sl])
                    return 0

                lax.fori_loop(0, DH // 64, vec, 0)
            wh.append(pltpu.async_copy(
                abufs[b], out_hbm.at[pl.ds(base + c * CCH, CCH)], wsems[b]))
        wh[0].wait()
        wh[1].wait()
        return 0

    lax.fori_loop(0, (TPW // CCH) // 2, pair, 0)


def _combine(ys, pos0, pos1, interpret=False):
    mesh = plsc.VectorSubcoreMesh(core_axis_name="c", subcore_axis_name="s")
    bufty = pltpu.VMEM((CCH, DH), jnp.float32)
    fn = pl.kernel(
        _combine_body,
        out_type=jax.ShapeDtypeStruct((N, DH), jnp.float32),
        mesh=mesh,
        scratch_types=[
            pltpu.VMEM((2, TPW), jnp.int32),
            bufty, bufty, bufty, bufty,
            pltpu.SemaphoreType.DMA,
            pltpu.SemaphoreType.DMA,
            pltpu.SemaphoreType.DMA,
            pltpu.SemaphoreType.DMA,
        ],
        interpret=interpret,
    )
    return fn(ys, pos0, pos1)


def _pipeline(x, w_gate, expert_w, expert_b, interpret=False):
    e0, e1, r0, r1, g0, g1, poff, be = _routing(x, w_gate,
                                                interpret=interpret)
    be = be.reshape(128)
    p0, p1, g0r, g1r = _posmap(e0, e1, r0, r1, g0, g1, poff,
                               interpret=interpret)
    p0, p1 = p0.reshape(N), p1.reshape(N)
    xs, gp = _dispatch(x, p0, p1, g0r, g1r, interpret=interpret)
    ys = _gmm(be, xs, gp, expert_w, expert_b, interpret=interpret)
    return _combine(ys, p0, p1, interpret=interpret)


def kernel(x, w_gate, w_noise, expert_w, expert_b):
    return _pipeline(x, w_gate, expert_w, expert_b)
